# baseline (speedup 1.0000x reference)
import functools

import jax
import jax.numpy as jnp
from jax.experimental import pallas as pl
from jax.experimental.pallas import tpu as pltpu

NUM_EXPERTS = 16
TOP_K = 2
AUX_COEFF = 0.01
BLOCK_T = 1024


def _router_kernel(x_ref, w_ref, ew_ref, ei_ref, aux_ref, acc_ref, *, n_tokens, n_blocks):
    i = pl.program_id(0)

    @pl.when(i == 0)
    def _init():
        acc_ref[...] = jnp.zeros_like(acc_ref)

    logits = jax.lax.dot_general(
        x_ref[...], w_ref[...],
        dimension_numbers=(((1,), (1,)), ((), ())),
        preferred_element_type=jnp.float32,
    )

    lane = jax.lax.broadcasted_iota(jnp.int32, logits.shape, 1)

    m1 = jnp.max(logits, axis=1, keepdims=True)
    i1 = jnp.argmax(logits, axis=1).astype(jnp.int32)
    is_top1 = lane == i1[:, None]
    masked = jnp.where(is_top1, -jnp.inf, logits)
    m2 = jnp.max(masked, axis=1, keepdims=True)
    i2 = jnp.argmax(masked, axis=1).astype(jnp.int32)

    e2 = jnp.exp(m2 - m1)
    denom2 = 1.0 + e2
    w1 = 1.0 / denom2
    w2 = e2 / denom2
    ew_ref[...] = jnp.concatenate([w1, w2], axis=1)
    ei_ref[...] = jnp.stack([i1, i2], axis=1)

    ex = jnp.exp(logits - m1)
    gates = ex / jnp.sum(ex, axis=1, keepdims=True)

    p_part = jnp.sum(gates, axis=0, keepdims=True)
    f_part = jnp.sum(is_top1.astype(jnp.float32), axis=0, keepdims=True)
    acc_ref[0:1, :] += f_part
    acc_ref[1:2, :] += p_part

    @pl.when(i == n_blocks - 1)
    def _finish():
        f = acc_ref[0:1, :] / n_tokens
        p = acc_ref[1:2, :] / n_tokens
        aux_ref[...] = (AUX_COEFF * NUM_EXPERTS * jnp.sum(f * p)).reshape(1, 1)


def kernel(x, W):
    n_tokens, d_model = x.shape
    n_blocks = n_tokens // BLOCK_T

    ew, ei, aux = pl.pallas_call(
        functools.partial(_router_kernel, n_tokens=n_tokens, n_blocks=n_blocks),
        grid=(n_blocks,),
        in_specs=[
            pl.BlockSpec((BLOCK_T, d_model), lambda i: (i, 0)),
            pl.BlockSpec((NUM_EXPERTS, d_model), lambda i: (0, 0)),
        ],
        out_specs=[
            pl.BlockSpec((BLOCK_T, TOP_K), lambda i: (i, 0)),
            pl.BlockSpec((BLOCK_T, TOP_K), lambda i: (i, 0)),
            pl.BlockSpec((1, 1), lambda i: (0, 0)),
        ],
        out_shape=[
            jax.ShapeDtypeStruct((n_tokens, TOP_K), jnp.float32),
            jax.ShapeDtypeStruct((n_tokens, TOP_K), jnp.int32),
            jax.ShapeDtypeStruct((1, 1), jnp.float32),
        ],
        scratch_shapes=[pltpu.VMEM((2, NUM_EXPERTS), jnp.float32)],
    )(x, W)
    return ew, ei, aux.reshape(())


# speedup vs baseline: 1.0210x; 1.0210x over previous
import functools

import jax
import jax.numpy as jnp
from jax.experimental import pallas as pl
from jax.experimental.pallas import tpu as pltpu

NUM_EXPERTS = 16
TOP_K = 2
AUX_COEFF = 0.01
BLOCK_T = 2048


def _router_kernel(x_ref, w_ref, ew_ref, ei_ref, aux_ref, acc_ref, *, n_tokens, n_blocks):
    i = pl.program_id(0)

    @pl.when(i == 0)
    def _init():
        acc_ref[...] = jnp.zeros_like(acc_ref)

    logits = jax.lax.dot_general(
        x_ref[...], w_ref[...],
        dimension_numbers=(((1,), (1,)), ((), ())),
        preferred_element_type=jnp.float32,
    )

    lane = jax.lax.broadcasted_iota(jnp.int32, logits.shape, 1)

    m1 = jnp.max(logits, axis=1, keepdims=True)
    i1 = jnp.argmax(logits, axis=1).astype(jnp.int32)
    is_top1 = lane == i1[:, None]
    masked = jnp.where(is_top1, -jnp.inf, logits)
    m2 = jnp.max(masked, axis=1, keepdims=True)
    i2 = jnp.argmax(masked, axis=1).astype(jnp.int32)

    e2 = jnp.exp(m2 - m1)
    denom2 = 1.0 + e2
    w1 = 1.0 / denom2
    w2 = e2 / denom2
    ew_ref[...] = jnp.concatenate([w1, w2], axis=1)
    ei_ref[...] = jnp.stack([i1, i2], axis=1)

    ex = jnp.exp(logits - m1)
    gates = ex / jnp.sum(ex, axis=1, keepdims=True)

    p_part = jnp.sum(gates, axis=0, keepdims=True)
    f_part = jnp.sum(is_top1.astype(jnp.float32), axis=0, keepdims=True)
    acc_ref[0:1, :] += f_part
    acc_ref[1:2, :] += p_part

    @pl.when(i == n_blocks - 1)
    def _finish():
        f = acc_ref[0:1, :] / n_tokens
        p = acc_ref[1:2, :] / n_tokens
        aux_ref[...] = (AUX_COEFF * NUM_EXPERTS * jnp.sum(f * p)).reshape(1, 1)


def kernel(x, W):
    n_tokens, d_model = x.shape
    n_blocks = n_tokens // BLOCK_T

    ew, ei, aux = pl.pallas_call(
        functools.partial(_router_kernel, n_tokens=n_tokens, n_blocks=n_blocks),
        grid=(n_blocks,),
        in_specs=[
            pl.BlockSpec((BLOCK_T, d_model), lambda i: (i, 0)),
            pl.BlockSpec((NUM_EXPERTS, d_model), lambda i: (0, 0)),
        ],
        out_specs=[
            pl.BlockSpec((BLOCK_T, TOP_K), lambda i: (i, 0)),
            pl.BlockSpec((BLOCK_T, TOP_K), lambda i: (i, 0)),
            pl.BlockSpec((1, 1), lambda i: (0, 0)),
        ],
        out_shape=[
            jax.ShapeDtypeStruct((n_tokens, TOP_K), jnp.float32),
            jax.ShapeDtypeStruct((n_tokens, TOP_K), jnp.int32),
            jax.ShapeDtypeStruct((1, 1), jnp.float32),
        ],
        scratch_shapes=[pltpu.VMEM((2, NUM_EXPERTS), jnp.float32)],
    )(x, W)
    return ew, ei, aux.reshape(())
